# baseline (speedup 1.0000x reference)
import functools

import jax
import jax.numpy as jnp
from jax import lax
from jax.experimental import pallas as pl
from jax.experimental.pallas import tpu as pltpu
from jax.experimental.pallas import tpu_sc as plsc

HEAD = 128
B, N, F = 8, 31, 8
NP1 = N + 1
TASKS = B * NP1 * NP1
ROWS = TASKS * NP1
DATA_TASKS = B * N * N
NZERO = TASKS - DATA_TASKS
L = 16
C = HEAD // L
NWORKERS = 32
DPW = -(-DATA_TASKS // NWORKERS)
ZPW = -(-NZERO // NWORKERS)
IW = N * F
TABLE_WORDS = 512 * HEAD

_GATHER_DN = lax.GatherDimensionNumbers(
    offset_dims=(), collapsed_slice_dims=(0,), start_index_map=(0,))


def _permute(x, idx):
    return lax.gather(x, idx[:, None], _GATHER_DN, slice_sizes=(1,),
                      mode=lax.GatherScatterMode.PROMISE_IN_BOUNDS)


def _lane_sum(x):
    for s in (8, 4, 2, 1):
        perm = jnp.bitwise_xor(jnp.arange(L, dtype=jnp.int32), s)
        x = x + _permute(x, perm)
    return x


def _sc_body(tfc_hbm, tab_hbm, gamma_hbm, beta_hbm, out_hbm,
             table_v, idx_v, obuf_v, zbuf_v, gb_v,
             sem_i, sem_o, sem_z):
    wid = lax.axis_index("s") * 2 + lax.axis_index("c")

    zero16 = jnp.zeros((L,), jnp.float32)

    def zrow(rr, c_):
        for c in range(C):
            zbuf_v[rr, pl.ds(16 * c, L)] = zero16
        return c_

    lax.fori_loop(0, NP1, zrow, 0)

    nz = jnp.minimum(ZPW, jnp.maximum(0, NZERO - wid * ZPW))

    def zfire(m, c_):
        z = wid * ZPW + m
        b = z // (NZERO // B)
        r = z % (NZERO // B)
        idr = jnp.where(r < NP1, r, (r - N) * NP1)
        row0 = (b * NP1 * NP1 + idr) * NP1
        pltpu.make_async_copy(zbuf_v, out_hbm.at[pl.ds(row0, NP1)],
                              sem_z).start()
        return c_

    lax.fori_loop(0, nz, zfire, 0)

    pltpu.sync_copy(tab_hbm, table_v)
    pltpu.sync_copy(gamma_hbm, gb_v.at[pl.ds(0, HEAD)])
    pltpu.sync_copy(beta_hbm, gb_v.at[pl.ds(HEAD, HEAD)])

    gvecs = [gb_v[pl.ds(16 * c, L)] for c in range(C)]
    bvecs = [gb_v[pl.ds(HEAD + 16 * c, L)] for c in range(C)]

    base_d = wid * DPW
    nq = jnp.minimum(DPW, DATA_TASKS - base_d)

    pltpu.make_async_copy(tfc_hbm.at[pl.ds(base_d * IW, IW)],
                          idx_v.at[pl.ds(0, IW)], sem_i).start()

    for s_ in range(2):
        for c in range(C):
            obuf_v[s_, 0, pl.ds(16 * c, L)] = zero16

    def dtask(q, c_):
        slot = q % 2
        d = base_d + q
        pltpu.make_async_copy(tfc_hbm.at[pl.ds(0, IW)],
                              idx_v.at[pl.ds(0, IW)], sem_i).wait()

        @pl.when(q + 1 < nq)
        def _prefetch():
            pltpu.make_async_copy(tfc_hbm.at[pl.ds((d + 1) * IW, IW)],
                                  idx_v.at[pl.ds((1 - slot) * 256, IW)],
                                  sem_i).start()

        @pl.when(q >= 2)
        def _reclaim():
            pltpu.make_async_copy(out_hbm.at[pl.ds(0, NP1)],
                                  obuf_v.at[slot], sem_o).wait()

        b = d // (N * N)
        r = d % (N * N)
        i = r // N
        j = r % N
        row0 = (b * NP1 * NP1 + (i + 1) * NP1 + (j + 1)) * NP1

        @plsc.parallel_loop(0, N, unroll=2, carry=(tuple(gvecs), tuple(bvecs)))
        def pos(k, gb):
            gv, bv = gb
            idx_vec = idx_v[pl.ds(slot * 256 + k * F, L)]
            accs = [None] * C
            for f in range(F):
                xf = idx_vec[f]
                base = xf * HEAD
                for c in range(C):
                    v = table_v[pl.ds(base + 16 * c, L)]
                    accs[c] = v if f == 0 else accs[c] + v
            tot = accs[0]
            sq = accs[0] * accs[0]
            for c in range(1, C):
                tot = tot + accs[c]
                sq = sq + accs[c] * accs[c]
            mv = _lane_sum(tot) * (1.0 / HEAD)
            vv = _lane_sum(sq) * (1.0 / HEAD) - mv * mv + 1e-5
            v0 = vv[0]
            ib = lax.bitcast_convert_type(v0, jnp.int32)
            yb = jnp.int32(0x5F3759DF) - lax.shift_right_logical(ib, 1)
            y0 = lax.bitcast_convert_type(yb, jnp.float32)
            for _ in range(2):
                y0 = y0 * (1.5 - 0.5 * v0 * y0 * y0)
            y = jnp.full((L,), y0, jnp.float32)
            for c in range(C):
                h = (accs[c] - mv) * y * gv[c] + bv[c]
                obuf_v[slot, k + 1, pl.ds(16 * c, L)] = h
            return gb

        pltpu.make_async_copy(obuf_v.at[slot], out_hbm.at[pl.ds(row0, NP1)],
                              sem_o).start()
        return c_

    lax.fori_loop(0, nq, dtask, 0)

    pltpu.make_async_copy(out_hbm.at[pl.ds(0, NP1)], obuf_v.at[0], sem_o).wait()
    pltpu.make_async_copy(out_hbm.at[pl.ds(0, NP1)], obuf_v.at[1], sem_o).wait()

    def zdrain(m, c_):
        pltpu.make_async_copy(out_hbm.at[pl.ds(0, NP1)], zbuf_v, sem_z).wait()
        return c_

    lax.fori_loop(0, nz, zdrain, 0)


@functools.partial(jax.jit)
def kernel(triplet_feat_cate, table, gamma, beta):
    starts = (jnp.arange(F, dtype=jnp.int32) * 64)
    tfc = (triplet_feat_cate + starts).reshape(DATA_TASKS * IW)
    tab = table.reshape(TABLE_WORDS)

    mesh = plsc.VectorSubcoreMesh(core_axis_name="c", subcore_axis_name="s")
    call = functools.partial(
        pl.kernel,
        mesh=mesh,
        out_type=jax.ShapeDtypeStruct((ROWS, HEAD), jnp.float32),
        scratch_types=[
            pltpu.VMEM((TABLE_WORDS,), jnp.float32),
            pltpu.VMEM((512,), jnp.int32),
            pltpu.VMEM((2, NP1, HEAD), jnp.float32),
            pltpu.VMEM((NP1, HEAD), jnp.float32),
            pltpu.VMEM((2 * HEAD,), jnp.float32),
            pltpu.SemaphoreType.DMA,
            pltpu.SemaphoreType.DMA,
            pltpu.SemaphoreType.DMA,
        ],
    )(_sc_body)
    out = call(tfc, tab, gamma, beta)
    return out.reshape(B, NP1, NP1, NP1, HEAD)


# speedup vs baseline: 1.0390x; 1.0390x over previous
import functools

import jax
import jax.numpy as jnp
from jax import lax
from jax.experimental import pallas as pl
from jax.experimental.pallas import tpu as pltpu
from jax.experimental.pallas import tpu_sc as plsc

HEAD = 128
B, N, F = 8, 31, 8
NP1 = N + 1
TASKS = B * NP1 * NP1
ROWS = TASKS * NP1
DATA_TASKS = B * N * N
NZERO = TASKS - DATA_TASKS
L = 16
C = HEAD // L
NWORKERS = 32
DPW = -(-DATA_TASKS // NWORKERS)
ZPW = -(-NZERO // NWORKERS)
IW = N * F
TABLE_WORDS = 512 * HEAD

_GATHER_DN = lax.GatherDimensionNumbers(
    offset_dims=(), collapsed_slice_dims=(0,), start_index_map=(0,))


def _permute(x, idx):
    return lax.gather(x, idx[:, None], _GATHER_DN, slice_sizes=(1,),
                      mode=lax.GatherScatterMode.PROMISE_IN_BOUNDS)


def _lane_sum(x):
    for s in (8, 4, 2, 1):
        perm = jnp.bitwise_xor(jnp.arange(L, dtype=jnp.int32), s)
        x = x + _permute(x, perm)
    return x


def _sc_body(tfc_hbm, tab_hbm, gamma_hbm, beta_hbm, out_hbm,
             table_v, idx_v, obuf_v, zbuf_v, gb_v,
             sem_i, sem_o, sem_z):
    wid = lax.axis_index("s") * 2 + lax.axis_index("c")

    zero16 = jnp.zeros((L,), jnp.float32)

    def zrow(rr, c_):
        for c in range(C):
            zbuf_v[rr, pl.ds(16 * c, L)] = zero16
        return c_

    lax.fori_loop(0, NP1, zrow, 0)

    nz = jnp.minimum(ZPW, jnp.maximum(0, NZERO - wid * ZPW))

    def zfire(m, c_):
        z = wid * ZPW + m
        b = z // (NZERO // B)
        r = z % (NZERO // B)
        idr = jnp.where(r < NP1, r, (r - N) * NP1)
        row0 = (b * NP1 * NP1 + idr) * NP1
        pltpu.make_async_copy(zbuf_v, out_hbm.at[pl.ds(row0, NP1)],
                              sem_z).start()
        return c_

    lax.fori_loop(0, nz, zfire, 0)

    pltpu.sync_copy(tab_hbm, table_v)
    pltpu.sync_copy(gamma_hbm, gb_v.at[pl.ds(0, HEAD)])
    pltpu.sync_copy(beta_hbm, gb_v.at[pl.ds(HEAD, HEAD)])

    gvecs = [gb_v[pl.ds(16 * c, L)] for c in range(C)]
    bvecs = [gb_v[pl.ds(HEAD + 16 * c, L)] for c in range(C)]

    base_d = wid * DPW
    nq = jnp.minimum(DPW, DATA_TASKS - base_d)

    pltpu.make_async_copy(tfc_hbm.at[pl.ds(base_d * IW, IW)],
                          idx_v.at[pl.ds(0, IW)], sem_i).start()

    for s_ in range(2):
        for c in range(C):
            obuf_v[s_, 0, pl.ds(16 * c, L)] = zero16

    def dtask(q, c_):
        slot = q % 2
        d = base_d + q
        pltpu.make_async_copy(tfc_hbm.at[pl.ds(0, IW)],
                              idx_v.at[pl.ds(0, IW)], sem_i).wait()

        @pl.when(q + 1 < nq)
        def _prefetch():
            pltpu.make_async_copy(tfc_hbm.at[pl.ds((d + 1) * IW, IW)],
                                  idx_v.at[pl.ds((1 - slot) * 256, IW)],
                                  sem_i).start()

        @pl.when(q >= 2)
        def _reclaim():
            pltpu.make_async_copy(out_hbm.at[pl.ds(0, NP1)],
                                  obuf_v.at[slot], sem_o).wait()

        b = d // (N * N)
        r = d % (N * N)
        i = r // N
        j = r % N
        row0 = (b * NP1 * NP1 + (i + 1) * NP1 + (j + 1)) * NP1

        @plsc.parallel_loop(0, N, unroll=2, carry=(tuple(gvecs), tuple(bvecs)))
        def pos(k, gb):
            gv, bv = gb
            idx_vec = idx_v[pl.ds(slot * 256 + k * F, L)]
            accs = [None] * C
            for f in range(F):
                xf = idx_vec[f] + f * 64
                base = xf * HEAD
                for c in range(C):
                    v = table_v[pl.ds(base + 16 * c, L)]
                    accs[c] = v if f == 0 else accs[c] + v
            tot = accs[0]
            sq = accs[0] * accs[0]
            for c in range(1, C):
                tot = tot + accs[c]
                sq = sq + accs[c] * accs[c]
            mv = _lane_sum(tot) * (1.0 / HEAD)
            vv = _lane_sum(sq) * (1.0 / HEAD) - mv * mv + 1e-5
            v0 = vv[0]
            ib = lax.bitcast_convert_type(v0, jnp.int32)
            yb = jnp.int32(0x5F3759DF) - lax.shift_right_logical(ib, 1)
            y0 = lax.bitcast_convert_type(yb, jnp.float32)
            for _ in range(2):
                y0 = y0 * (1.5 - 0.5 * v0 * y0 * y0)
            y = jnp.full((L,), y0, jnp.float32)
            for c in range(C):
                h = (accs[c] - mv) * y * gv[c] + bv[c]
                obuf_v[slot, k + 1, pl.ds(16 * c, L)] = h
            return gb

        pltpu.make_async_copy(obuf_v.at[slot], out_hbm.at[pl.ds(row0, NP1)],
                              sem_o).start()
        return c_

    lax.fori_loop(0, nq, dtask, 0)

    pltpu.make_async_copy(out_hbm.at[pl.ds(0, NP1)], obuf_v.at[0], sem_o).wait()
    pltpu.make_async_copy(out_hbm.at[pl.ds(0, NP1)], obuf_v.at[1], sem_o).wait()

    def zdrain(m, c_):
        pltpu.make_async_copy(out_hbm.at[pl.ds(0, NP1)], zbuf_v, sem_z).wait()
        return c_

    lax.fori_loop(0, nz, zdrain, 0)


@functools.partial(jax.jit)
def kernel(triplet_feat_cate, table, gamma, beta):
    tfc = triplet_feat_cate.reshape(DATA_TASKS * IW)
    tab = table.reshape(TABLE_WORDS)

    mesh = plsc.VectorSubcoreMesh(core_axis_name="c", subcore_axis_name="s")
    call = functools.partial(
        pl.kernel,
        mesh=mesh,
        out_type=jax.ShapeDtypeStruct((ROWS, HEAD), jnp.float32),
        scratch_types=[
            pltpu.VMEM((TABLE_WORDS,), jnp.float32),
            pltpu.VMEM((512,), jnp.int32),
            pltpu.VMEM((2, NP1, HEAD), jnp.float32),
            pltpu.VMEM((NP1, HEAD), jnp.float32),
            pltpu.VMEM((2 * HEAD,), jnp.float32),
            pltpu.SemaphoreType.DMA,
            pltpu.SemaphoreType.DMA,
            pltpu.SemaphoreType.DMA,
        ],
    )(_sc_body)
    out = call(tfc, tab, gamma, beta)
    return out.reshape(B, NP1, NP1, NP1, HEAD)
